# baseline (speedup 1.0000x reference)
import functools

import jax
import jax.numpy as jnp
from jax import lax
from jax.experimental import pallas as pl
from jax.experimental.pallas import tpu as pltpu
from jax.experimental.pallas import tpu_sc as plsc


def _fuse_body(table_ref, w_ref, b_ref, f_ref):
    f_ref[...] = jnp.dot(table_ref[...], w_ref[...],
                         preferred_element_type=jnp.float32) + b_ref[...]


def _fuse_table(table, W16, b16):
    V = table.shape[0]
    return pl.pallas_call(
        _fuse_body,
        out_shape=jax.ShapeDtypeStruct((V, 16), jnp.float32),
    )(table, W16, b16)


def _make_gather(B, DP, DI, chunk):
    info = plsc.get_sparse_core_info()
    nw = info.num_cores * info.num_subcores
    b_per_w = B // nw
    n_chunks = b_per_w // chunk
    mesh = plsc.VectorSubcoreMesh(core_axis_name="c", subcore_axis_name="s")

    @functools.partial(
        pl.kernel,
        mesh=mesh,
        compiler_params=pltpu.CompilerParams(use_tc_tiling_on_sc=False),
        out_type=[
            jax.ShapeDtypeStruct((B, DP), jnp.float32),
            jax.ShapeDtypeStruct((B, DI), jnp.float32),
        ],
        scratch_types=[
            pltpu.VMEM((chunk,), jnp.int32),
            pltpu.VMEM((chunk, 16), jnp.float32),
            pltpu.SemaphoreType.DMA,
        ],
    )
    def gather(f_hbm, idx_hbm, outp_hbm, outi_hbm, idx_v, r_v, sem):
        wid = lax.axis_index("s") * info.num_cores + lax.axis_index("c")
        base = wid * b_per_w

        def chunk_body(i, carry):
            off = base + i * chunk
            pltpu.sync_copy(idx_hbm.at[pl.ds(off, chunk)], idx_v)
            pltpu.async_copy(f_hbm.at[idx_v], r_v, sem).wait()
            pltpu.sync_copy(r_v.at[:, pl.ds(0, DP)], outp_hbm.at[pl.ds(off, chunk)])
            pltpu.sync_copy(r_v.at[:, pl.ds(8, DI)], outi_hbm.at[pl.ds(off, chunk)])
            return carry

        lax.fori_loop(0, n_chunks, chunk_body, 0)

    return gather


def kernel(x, table, Wp, bp, Wi, bi):
    Bb, L = x.shape
    B = Bb * L
    DP = Wp.shape[1]
    DI = Wi.shape[1]
    E = table.shape[1]
    W16 = jnp.zeros((E, 16), jnp.float32)
    W16 = W16.at[:, 0:DP].set(Wp).at[:, 8:8 + DI].set(Wi)
    b16 = jnp.zeros((1, 16), jnp.float32)
    b16 = b16.at[0, 0:DP].set(bp).at[0, 8:8 + DI].set(bi)
    f16 = _fuse_table(table, W16, b16)
    gather = _make_gather(B, DP, DI, chunk=4096)
    outp, outi = gather(f16, x.reshape(B))
    return outp.reshape(Bb, L, DP), outi.reshape(Bb, L, DI)


# speedup vs baseline: 3.0035x; 3.0035x over previous
import functools

import jax
import jax.numpy as jnp
from jax import lax
from jax.experimental import pallas as pl
from jax.experimental.pallas import tpu as pltpu
from jax.experimental.pallas import tpu_sc as plsc


def _fuse_body(table_ref, w_ref, b_ref, f_ref):
    f_ref[...] = jnp.dot(table_ref[...], w_ref[...],
                         preferred_element_type=jnp.float32) + b_ref[...]


def _fuse_table(table, W16, b16):
    V = table.shape[0]
    return pl.pallas_call(
        _fuse_body,
        out_shape=jax.ShapeDtypeStruct((V, 16), jnp.float32),
    )(table, W16, b16)


def _make_lookup(B, V, DP, DI, chunk):
    info = plsc.get_sparse_core_info()
    nw = info.num_cores * info.num_subcores
    L = info.num_lanes
    b_per_w = B // nw
    n_chunks = b_per_w // chunk
    n_groups = chunk // L
    mesh = plsc.VectorSubcoreMesh(core_axis_name="c", subcore_axis_name="s")

    @functools.partial(
        pl.kernel,
        mesh=mesh,
        compiler_params=pltpu.CompilerParams(use_tc_tiling_on_sc=False,
                                             needs_layout_passes=False),
        out_type=[
            jax.ShapeDtypeStruct((B * DP,), jnp.float32),
            jax.ShapeDtypeStruct((B * DI,), jnp.float32),
        ],
        scratch_types=[
            pltpu.VMEM((V, 16), jnp.float32),
            pltpu.VMEM((chunk,), jnp.int32),
            pltpu.VMEM((chunk * DP,), jnp.float32),
            pltpu.VMEM((chunk * DI,), jnp.float32),
        ],
    )
    def lookup(f_hbm, idx_hbm, outp_hbm, outi_hbm, tab_v, idx_v, op_v, oi_v):
        wid = lax.axis_index("s") * info.num_cores + lax.axis_index("c")
        base = wid * b_per_w
        pltpu.sync_copy(f_hbm, tab_v)
        lanes = lax.iota(jnp.int32, L)

        def chunk_body(i, carry):
            off = base + i * chunk
            pltpu.sync_copy(idx_hbm.at[pl.ds(off, chunk)], idx_v)

            def group_body(g, carry2):
                t0 = g * L
                for k in range(DP):
                    pos = k * L + lanes
                    tvec = t0 + pos // DP
                    cvec = pos % DP
                    rows = plsc.load_gather(idx_v, [tvec])
                    vals = plsc.load_gather(tab_v, [rows, cvec])
                    op_v[pl.ds(g * (L * DP) + k * L, L)] = vals
                for k in range(DI):
                    pos = k * L + lanes
                    tvec = t0 + pos // DI
                    cvec = 8 + pos % DI
                    rows = plsc.load_gather(idx_v, [tvec])
                    vals = plsc.load_gather(tab_v, [rows, cvec])
                    oi_v[pl.ds(g * (L * DI) + k * L, L)] = vals
                return carry2

            lax.fori_loop(0, n_groups, group_body, 0)
            pltpu.sync_copy(op_v, outp_hbm.at[pl.ds(off * DP, chunk * DP)])
            pltpu.sync_copy(oi_v, outi_hbm.at[pl.ds(off * DI, chunk * DI)])
            return carry

        lax.fori_loop(0, n_chunks, chunk_body, 0)

    return lookup


def kernel(x, table, Wp, bp, Wi, bi):
    Bb, L = x.shape
    B = Bb * L
    DP = Wp.shape[1]
    DI = Wi.shape[1]
    E = table.shape[1]
    V = table.shape[0]
    W16 = jnp.zeros((E, 16), jnp.float32)
    W16 = W16.at[:, 0:DP].set(Wp).at[:, 8:8 + DI].set(Wi)
    b16 = jnp.zeros((1, 16), jnp.float32)
    b16 = b16.at[0, 0:DP].set(bp).at[0, 8:8 + DI].set(bi)
    f16 = _fuse_table(table, W16, b16)
    lookup = _make_lookup(B, V, DP, DI, chunk=4096)
    outp, outi = lookup(f16, x.reshape(B))
    return outp.reshape(Bb, L, DP), outi.reshape(Bb, L, DI)


# speedup vs baseline: 19.2387x; 6.4054x over previous
import functools

import jax
import jax.numpy as jnp
from jax import lax
from jax.experimental import pallas as pl
from jax.experimental.pallas import tpu as pltpu
from jax.experimental.pallas import tpu_sc as plsc


def _fuse_body(table_ref, w_ref, b_ref, f_ref):
    f_ref[...] = jnp.dot(table_ref[...], w_ref[...],
                         preferred_element_type=jnp.float32) + b_ref[...]


def _fuse_table(table, W16, b16):
    V = table.shape[0]
    return pl.pallas_call(
        _fuse_body,
        out_shape=jax.ShapeDtypeStruct((V, 16), jnp.float32),
    )(table, W16, b16)


def _make_lookup(B, V, DP, DI, NR, NC_TILES):
    info = plsc.get_sparse_core_info()
    nw = info.num_cores * info.num_subcores
    L = info.num_lanes
    tpw = NC_TILES // nw
    run = tpw * 8 * 128
    n_vec = run // L
    slab_p = B
    slab_i = DI * 128
    mesh = plsc.VectorSubcoreMesh(core_axis_name="c", subcore_axis_name="s")

    @functools.partial(
        pl.kernel,
        mesh=mesh,
        compiler_params=pltpu.CompilerParams(use_tc_tiling_on_sc=False,
                                             needs_layout_passes=False),
        out_type=[
            jax.ShapeDtypeStruct((B * DP,), jnp.float32),
            jax.ShapeDtypeStruct((B * DI,), jnp.float32),
        ],
        scratch_types=[
            pltpu.VMEM((V * 16,), jnp.float32),
            pltpu.VMEM((run,), jnp.int32),
            pltpu.VMEM((DP * run,), jnp.float32),
            pltpu.VMEM((8 * tpw * DI * 128,), jnp.float32),
        ],
    )
    def lookup(f_hbm, xq_hbm, qp_hbm, qi_hbm, tab_v, idx_v, pbuf, ibuf):
        wid = lax.axis_index("s") * info.num_cores + lax.axis_index("c")
        pltpu.sync_copy(f_hbm, tab_v)

        def r_body(R, carry):
            off = R * (NC_TILES * 1024) + wid * run
            pltpu.sync_copy(xq_hbm.at[pl.ds(off, run)], idx_v)

            def v_body(v, carry2):
                idxv = idx_v[pl.ds(v * L, L)]
                base = idxv << 4
                for d2 in range(DP):
                    vals = plsc.load_gather(tab_v, [base + d2])
                    pbuf[pl.ds(d2 * run + v * L, L)] = vals
                r = (v >> 3) & 7
                cl = v >> 6
                c16 = v & 7
                ioff = r * (tpw * slab_i) + cl * slab_i + c16 * L
                for d2 in range(DI):
                    vals = plsc.load_gather(tab_v, [base + (8 + d2)])
                    ibuf[pl.ds(ioff + d2 * 128, L)] = vals
                return carry2

            lax.fori_loop(0, n_vec, v_body, 0)
            for d2 in range(DP):
                pltpu.sync_copy(
                    pbuf.at[pl.ds(d2 * run, run)],
                    qp_hbm.at[pl.ds(d2 * slab_p + off, run)])
            for r in range(8):
                pltpu.sync_copy(
                    ibuf.at[pl.ds(r * (tpw * slab_i), tpw * slab_i)],
                    qi_hbm.at[pl.ds((R * 8 + r) * (NC_TILES * slab_i)
                                    + wid * (tpw * slab_i), tpw * slab_i)])
            return carry

        lax.fori_loop(0, NR, r_body, 0)

    return lookup


def kernel(x, table, Wp, bp, Wi, bi):
    Bb, Lp = x.shape
    B = Bb * Lp
    DP = Wp.shape[1]
    DI = Wi.shape[1]
    E = table.shape[1]
    V = table.shape[0]
    NR = Lp // 8
    NCT = Bb // 128
    W16 = jnp.zeros((E, 16), jnp.float32)
    W16 = W16.at[:, 0:DP].set(Wp).at[:, 8:8 + DI].set(Wi)
    b16 = jnp.zeros((1, 16), jnp.float32)
    b16 = b16.at[0, 0:DP].set(bp).at[0, 8:8 + DI].set(bi)
    f16 = _fuse_table(table, W16, b16)
    xq = x.reshape(NCT, 128, NR, 8).transpose(2, 0, 3, 1).reshape(B)
    lookup = _make_lookup(B, V, DP, DI, NR, NCT)
    qp, qi = lookup(f16.reshape(V * 16), xq)
    outp = (qp.reshape(DP, NR, NCT, 8, 128)
            .transpose(2, 4, 1, 3, 0).reshape(Bb, Lp, DP))
    outi = (qi.reshape(Lp, NCT, DI, 128)
            .transpose(1, 3, 0, 2).reshape(Bb, Lp, DI))
    return outp, outi
